# Initial kernel scaffold; baseline (speedup 1.0000x reference)
import jax
import jax.numpy as jnp
from jax.experimental import pallas as pl


def kernel(x, edge_index, edge_weight, W1, b1, g1, be1, W2, b2, g2, be2, W3, b3, g3, be3):
    raise NotImplementedError("write your pallas kernel here")


# speedup vs baseline: 8.4096x; 8.4096x over previous
import jax
import jax.numpy as jnp
from jax import lax
from jax.experimental import pallas as pl
from jax.experimental.pallas import tpu as pltpu
from jax.experimental.pallas import tpu_sc as plsc

N = 10000
E = 320000
D = 128
CHUNK = 128
NCHUNKS = E // CHUNK
NLANE = 16


def _sc_mesh():
    info = plsc.get_sparse_core_info()
    mesh = plsc.VectorSubcoreMesh(core_axis_name="c", subcore_axis_name="s")
    return info.num_cores, info.num_subcores, mesh


def _row_sliced(s, ns, fn):
    rps = ((N + ns - 1) // ns + 7) // 8 * 8
    last = N - (ns - 1) * rps

    @pl.when(s < ns - 1)
    def _():
        fn(pl.multiple_of(s * rps, 8), rps)

    @pl.when(s == ns - 1)
    def _():
        fn((ns - 1) * rps, last)


def _sc_deg(col, ew, zeros16):
    nc, ns, mesh = _sc_mesh()
    nw = nc * ns

    def body(col_hbm, ew_hbm, z_hbm, out_hbm, acc, colv, ewv, ew16):
        c = lax.axis_index("c")
        s = lax.axis_index("s")
        w = s * nc + c
        _row_sliced(s, ns, lambda o, n: pltpu.sync_copy(
            z_hbm.at[pl.ds(o, n)], acc.at[pl.ds(o, n)]))
        plsc.subcore_barrier()
        nch = (NCHUNKS - w + nw - 1) // nw

        def chunk_body(i, carry):
            base = pl.multiple_of((w + i * nw) * CHUNK, CHUNK)
            pltpu.sync_copy(col_hbm.at[pl.ds(base, CHUNK)], colv)
            pltpu.sync_copy(ew_hbm.at[pl.ds(base, CHUNK)], ewv)

            def splat(j, cc):
                ew16[j, :] = plsc.load_gather(ewv, [jnp.full((NLANE,), j, jnp.int32)])
                return cc

            lax.fori_loop(0, CHUNK, splat, 0, unroll=4)
            pltpu.sync_copy(ew16, acc.at[colv], add=True)
            return carry

        lax.fori_loop(0, nch, chunk_body, 0)
        plsc.subcore_barrier()
        _row_sliced(s, ns, lambda o, n: pltpu.sync_copy(
            acc.at[pl.ds(o, n)], out_hbm.at[c, pl.ds(o, n)]))

    return pl.kernel(
        body,
        compiler_params=pltpu.CompilerParams(
            needs_layout_passes=False, use_tc_tiling_on_sc=False),
        out_type=jax.ShapeDtypeStruct((nc, N, NLANE), jnp.float32),
        mesh=mesh,
        scratch_types=[
            pltpu.VMEM_SHARED((N, NLANE), jnp.float32),
            pltpu.VMEM((CHUNK,), jnp.int32),
            pltpu.VMEM((CHUNK,), jnp.float32),
            pltpu.VMEM((CHUNK, NLANE), jnp.float32),
        ],
    )(col, ew, zeros16)


def _sc_agg(h, row, col, ew, dinv, zeros):
    nc, ns, mesh = _sc_mesh()
    nw = nc * ns

    def body(h_hbm, row_hbm, col_hbm, ew_hbm, dinv_hbm, z_hbm, out_hbm,
             acc, dinvv, rowv, colv, ewv, sbuf, rowsv):
        c = lax.axis_index("c")
        s = lax.axis_index("s")
        w = s * nc + c
        _row_sliced(s, ns, lambda o, n: pltpu.sync_copy(
            z_hbm.at[pl.ds(o, n)], acc.at[pl.ds(o, n)]))
        pltpu.sync_copy(dinv_hbm, dinvv)
        plsc.subcore_barrier()
        nch = (NCHUNKS - w + nw - 1) // nw

        def chunk_body(i, carry):
            base = pl.multiple_of((w + i * nw) * CHUNK, CHUNK)
            pltpu.sync_copy(row_hbm.at[pl.ds(base, CHUNK)], rowv)
            pltpu.sync_copy(col_hbm.at[pl.ds(base, CHUNK)], colv)
            pltpu.sync_copy(ew_hbm.at[pl.ds(base, CHUNK)], ewv)
            pltpu.sync_copy(h_hbm.at[rowv], rowsv)

            def pre(k, cc):
                r16 = rowv[pl.ds(k * NLANE, NLANE)]
                d16 = plsc.load_gather(dinvv, [r16])
                e16 = ewv[pl.ds(k * NLANE, NLANE)]
                sbuf[pl.ds(k * NLANE, NLANE)] = d16 * e16
                return cc

            lax.fori_loop(0, CHUNK // NLANE, pre, 0, unroll=True)

            def scale(j, cc):
                sp = plsc.load_gather(sbuf, [jnp.full((NLANE,), j, jnp.int32)])
                for f in range(D // NLANE):
                    sl = pl.ds(f * NLANE, NLANE)
                    rowsv[j, sl] = rowsv[j, sl] * sp
                return cc

            lax.fori_loop(0, CHUNK, scale, 0, unroll=2)
            pltpu.sync_copy(rowsv, acc.at[colv], add=True)
            return carry

        lax.fori_loop(0, nch, chunk_body, 0)
        plsc.subcore_barrier()
        _row_sliced(s, ns, lambda o, n: pltpu.sync_copy(
            acc.at[pl.ds(o, n)], out_hbm.at[c, pl.ds(o, n)]))

    return pl.kernel(
        body,
        compiler_params=pltpu.CompilerParams(needs_layout_passes=False),
        out_type=jax.ShapeDtypeStruct((nc, N, D), jnp.float32),
        mesh=mesh,
        scratch_types=[
            pltpu.VMEM_SHARED((N, D), jnp.float32),
            pltpu.VMEM((N,), jnp.float32),
            pltpu.VMEM((CHUNK,), jnp.int32),
            pltpu.VMEM((CHUNK,), jnp.int32),
            pltpu.VMEM((CHUNK,), jnp.float32),
            pltpu.VMEM((CHUNK,), jnp.float32),
            pltpu.VMEM((CHUNK, D), jnp.float32),
        ],
    )(h, row, col, ew, dinv, zeros)


def _tc_matmul(x, W):
    def body(x_ref, w_ref, o_ref):
        o_ref[:] = jnp.dot(x_ref[:], w_ref[:], preferred_element_type=jnp.float32)

    return pl.pallas_call(
        body, out_shape=jax.ShapeDtypeStruct((N, D), jnp.float32))(x, W)


def _tc_dinv(degp):
    def body(d_ref, o_ref):
        deg = d_ref[0, :, 0:1] + d_ref[1, :, 0:1] + 1.0
        o_ref[:] = lax.rsqrt(jnp.maximum(deg, 1e-12))

    return pl.pallas_call(
        body, out_shape=jax.ShapeDtypeStruct((N, 1), jnp.float32))(degp)


def _tc_layer(aggp, h, x0, dinv, b, g, be, W, relu):
    def body(a_ref, h_ref, x0_ref, di_ref, b_ref, g_ref, be_ref, *rest):
        if W is None:
            (o_ref,) = rest
        else:
            w_ref, o_ref = rest
        di = di_ref[:]
        o = di * (a_ref[0] + a_ref[1]) + (di * di) * h_ref[:] + b_ref[:] + x0_ref[:]
        if relu:
            o = jnp.maximum(o, 0.0)
        mu = jnp.mean(o, axis=0, keepdims=True)
        xc = o - mu
        var = jnp.mean(xc * xc, axis=0, keepdims=True)
        xn = xc * lax.rsqrt(var + 1e-5) * g_ref[:] + be_ref[:]
        if W is None:
            o_ref[:] = xn
        else:
            o_ref[:] = jnp.dot(xn, w_ref[:], preferred_element_type=jnp.float32)

    args = [aggp, h, x0, dinv, b.reshape(1, D), g.reshape(1, D), be.reshape(1, D)]
    if W is not None:
        args.append(W)
    return pl.pallas_call(
        body, out_shape=jax.ShapeDtypeStruct((N, D), jnp.float32))(*args)


def kernel(x, edge_index, edge_weight, W1, b1, g1, be1, W2, b2, g2, be2,
           W3, b3, g3, be3):
    x = x.astype(jnp.float32)
    row = edge_index[0]
    col = edge_index[1]
    ew = edge_weight.astype(jnp.float32)
    zeros16 = jnp.zeros((N, NLANE), jnp.float32)
    zeros = jnp.zeros((N, D), jnp.float32)

    degp = _sc_deg(col, ew, zeros16)
    h1 = _tc_matmul(x, W1)
    dinv = _tc_dinv(degp)
    dinv_flat = dinv.reshape(N)

    agg1 = _sc_agg(h1, row, col, ew, dinv_flat, zeros)
    h2 = _tc_layer(agg1, h1, x, dinv, b1, g1, be1, W2, relu=True)
    agg2 = _sc_agg(h2, row, col, ew, dinv_flat, zeros)
    h3 = _tc_layer(agg2, h2, x, dinv, b2, g2, be2, W3, relu=True)
    agg3 = _sc_agg(h3, row, col, ew, dinv_flat, zeros)
    out = _tc_layer(agg3, h3, x, dinv, b3, g3, be3, None, relu=False)
    return out
